# baseline (speedup 1.0000x reference)
import functools

import jax
import jax.numpy as jnp
from jax import lax
from jax.experimental import pallas as pl
from jax.experimental.pallas import tpu as pltpu
from jax.experimental.pallas import tpu_sc as plsc

_S = 8192
_D = 1024
_NC = 2
_NS = 16
_NW = _NC * _NS
_ROWS_PER_W = _S // _NW

_mesh = plsc.VectorSubcoreMesh(core_axis_name="c", subcore_axis_name="s")


@functools.partial(
    pl.kernel,
    mesh=_mesh,
    out_type=jax.ShapeDtypeStruct((_S, _D), jnp.float32),
)
def _embed_copy(table_hbm, out_hbm):
    wid = lax.axis_index("s") * _NC + lax.axis_index("c")
    base = wid * _ROWS_PER_W
    pltpu.sync_copy(table_hbm.at[pl.ds(base, _ROWS_PER_W)],
                    out_hbm.at[pl.ds(base, _ROWS_PER_W)])


def kernel(x, table):
    del x
    return _embed_copy(table)[None]


# speedup vs baseline: 24.3858x; 24.3858x over previous
import functools

import jax
import jax.numpy as jnp
from jax import lax
from jax.experimental import pallas as pl
from jax.experimental.pallas import tpu as pltpu
from jax.experimental.pallas import tpu_sc as plsc

_S = 8192
_D = 1024
_NC = 2
_NS = 16
_NW = _NC * _NS
_ROWS_PER_W = _S // _NW

_CHUNK = 32
_NCHUNK = _ROWS_PER_W // _CHUNK

_mesh = plsc.VectorSubcoreMesh(core_axis_name="c", subcore_axis_name="s")


@functools.partial(
    pl.kernel,
    mesh=_mesh,
    out_type=jax.ShapeDtypeStruct((_S, _D), jnp.float32),
    scratch_types=[
        pltpu.VMEM((2, _CHUNK, _D), jnp.float32),
        pltpu.SemaphoreType.DMA,
        pltpu.SemaphoreType.DMA,
        pltpu.SemaphoreType.DMA,
        pltpu.SemaphoreType.DMA,
    ],
)
def _embed_copy(table_hbm, out_hbm, buf, si0, si1, so0, so1):
    wid = lax.axis_index("s") * _NC + lax.axis_index("c")
    base = wid * _ROWS_PER_W
    sin = (si0, si1)
    sout = (so0, so1)

    def in_copy(i):
        return pltpu.async_copy(
            table_hbm.at[pl.ds(base + i * _CHUNK, _CHUNK)],
            buf.at[i % 2], sin[i % 2])

    def out_copy(i):
        return pltpu.async_copy(
            buf.at[i % 2],
            out_hbm.at[pl.ds(base + i * _CHUNK, _CHUNK)],
            sout[i % 2])

    hin = [None] * _NCHUNK
    hout = [None] * _NCHUNK
    hin[0] = in_copy(0)
    for i in range(_NCHUNK):
        if i + 1 < _NCHUNK:
            if i + 1 >= 2:
                hout[i - 1].wait()
            hin[i + 1] = in_copy(i + 1)
        hin[i].wait()
        hout[i] = out_copy(i)
    hout[-2].wait()
    hout[-1].wait()


def kernel(x, table):
    del x
    return _embed_copy(table)[None]
